# baseline (speedup 1.0000x reference)
import functools

import jax
import jax.numpy as jnp
from jax import lax
from jax.experimental import pallas as pl
from jax.experimental.pallas import tpu as pltpu
from jax.experimental.pallas import tpu_sc as plsc

B = 1024
KB = 8192
LANES = 128
BIG = 3.0e38

K_TOTAL = 100000
K_TC = 10 * KB
NW = 32
L = 16
NQ = 8
K_SC_REAL = K_TOTAL - K_TC
CH = -(-K_SC_REAL // (NW * L)) * L


def _sc_body(tx_hbm, ty_hbm, qx_hbm, qy_hbm, out_hbm, av, bv, cv, qxv, qyv,
             ov, tt):
    cid = lax.axis_index("c")
    sid = lax.axis_index("s")
    wid = sid * 2 + cid
    base = jnp.minimum(wid * CH, K_SC_REAL - CH)
    pltpu.sync_copy(tx_hbm.at[pl.ds(base, CH)], av)
    pltpu.sync_copy(ty_hbm.at[pl.ds(base, CH)], bv)
    pltpu.sync_copy(qx_hbm, qxv)
    pltpu.sync_copy(qy_hbm, qyv)

    def pre(i, carry):
        sl = pl.ds(i * L, L)
        tx = av[sl]
        ty = bv[sl]
        cv[sl] = tx * tx + ty * ty
        av[sl] = -2.0 * tx
        bv[sl] = -2.0 * ty
        return carry

    lax.fori_loop(0, CH // L, pre, 0, unroll=2)

    def qloop(qg, carry):
        qb = qg * L
        qxvec = qxv[pl.ds(qb, L)]
        qyvec = qyv[pl.ds(qb, L)]
        for sub in range(L // NQ):
            qxs = [qxvec[sub * NQ + u] for u in range(NQ)]
            qys = [qyvec[sub * NQ + u] for u in range(NQ)]

            def inner(j, accs):
                sl = pl.ds(j * L, L)
                a = av[sl]
                b = bv[sl]
                c = cv[sl]
                return tuple(
                    jnp.minimum(accs[u], c + qxs[u] * a + qys[u] * b)
                    for u in range(NQ))

            init = tuple(jnp.full((L,), BIG, jnp.float32) for _ in range(NQ))
            accs = lax.fori_loop(0, CH // L, inner, init, unroll=2)
            for u in range(NQ):
                tt[pl.ds((sub * NQ + u) * L, L)] = accs[u]
        rows = lax.iota(jnp.int32, L) * L
        mv = plsc.load_gather(tt, [rows])
        for j in range(1, L):
            mv = jnp.minimum(mv, plsc.load_gather(tt, [rows + j]))
        ov[pl.ds(qb, L)] = mv
        return carry

    lax.fori_loop(0, B // L, qloop, 0)
    pltpu.sync_copy(ov, out_hbm.at[wid])


def _sc_partial_min(tx_sc, ty_sc, qx, qy):
    mesh = plsc.VectorSubcoreMesh(
        core_axis_name="c", subcore_axis_name="s", num_cores=2,
        num_subcores=16)
    return pl.kernel(
        _sc_body,
        out_type=jax.ShapeDtypeStruct((NW, B), jnp.float32),
        mesh=mesh,
        compiler_params=pltpu.CompilerParams(needs_layout_passes=False),
        scratch_types=[
            pltpu.VMEM((CH,), jnp.float32),
            pltpu.VMEM((CH,), jnp.float32),
            pltpu.VMEM((CH,), jnp.float32),
            pltpu.VMEM((B,), jnp.float32),
            pltpu.VMEM((B,), jnp.float32),
            pltpu.VMEM((B,), jnp.float32),
            pltpu.VMEM((L * L,), jnp.float32),
        ],
    )(tx_sc, ty_sc, qx, qy)


def _tc_scan_body(cc_ref, tx_ref, ty_ref, sf_ref, w1_ref, b1_ref, out_ref,
                  h0_ref, acc_ref, *, nsteps):
    i = pl.program_id(0)
    qx = cc_ref[:, 0:1]
    qy = cc_ref[:, 1:2]
    ax = -2.0 * qx
    ay = -2.0 * qy

    tx = tx_ref[:].reshape(1, KB)
    ty = ty_ref[:].reshape(1, KB)
    c = tx * tx + ty * ty
    ones = jnp.ones_like(qx)
    zeros8 = jnp.zeros((B, 5), jnp.float32)
    amat = jnp.concatenate([ax, ay, ones, zeros8], axis=1)
    tmat = jnp.concatenate(
        [tx, ty, c, jnp.zeros((5, KB), jnp.float32)], axis=0)
    p = jnp.dot(amat, tmat, preferred_element_type=jnp.float32,
                precision=lax.Precision.DEFAULT)

    m = p[:, 0:LANES]
    for g in range(1, KB // LANES):
        m = jnp.minimum(m, p[:, g * LANES:(g + 1) * LANES])

    @pl.when(i == 0)
    def _():
        acc_ref[...] = m

    @pl.when(i > 0)
    def _():
        acc_ref[...] = jnp.minimum(acc_ref[...], m)

    @pl.when(i == nsteps - 1)
    def _():
        out_ref[...] = jnp.min(acc_ref[...], axis=1, keepdims=True)
        h0_ref[...] = jnp.dot(sf_ref[...], w1_ref[0:128, :],
                              preferred_element_type=jnp.float32) + b1_ref[...]


def _merge_body(tcm_ref, h0_ref, scp_ref, cc_ref, w1_ref, w2_ref,
                b2_ref, out_ref):
    qx = cc_ref[:, 0:1]
    qy = cc_ref[:, 1:2]
    q2 = qx * qx + qy * qy
    scm = jnp.min(jnp.transpose(scp_ref[...]), axis=1, keepdims=True)
    d2 = jnp.minimum(tcm_ref[...], scm) + q2
    dist = jnp.sqrt(jnp.maximum(d2, 1e-12))
    h = h0_ref[...] + dist * w1_ref[128:129, :]
    h = jnp.maximum(h, 0.0)
    z = jnp.dot(h, w2_ref[...], preferred_element_type=jnp.float32)
    out_ref[...] = jax.nn.sigmoid(z + b2_ref[...])


def kernel(spatial_features, current_coords, training_coords, W1, b1, W2, b2):
    scp = _sc_partial_min(training_coords[K_TC:, 0],
                          training_coords[K_TC:, 1],
                          current_coords[:, 0], current_coords[:, 1])

    tx = training_coords[:K_TC, 0]
    ty = training_coords[:K_TC, 1]

    nsteps = K_TC // KB
    tcm, h0 = pl.pallas_call(
        functools.partial(_tc_scan_body, nsteps=nsteps),
        grid=(nsteps,),
        in_specs=[
            pl.BlockSpec((B, 2), lambda i: (0, 0)),
            pl.BlockSpec((KB,), lambda i: (i,)),
            pl.BlockSpec((KB,), lambda i: (i,)),
            pl.BlockSpec((B, 128), lambda i: (0, 0)),
            pl.BlockSpec((129, 64), lambda i: (0, 0)),
            pl.BlockSpec((1, 64), lambda i: (0, 0)),
        ],
        out_specs=[
            pl.BlockSpec((B, 1), lambda i: (0, 0)),
            pl.BlockSpec((B, 64), lambda i: (0, 0)),
        ],
        out_shape=[
            jax.ShapeDtypeStruct((B, 1), jnp.float32),
            jax.ShapeDtypeStruct((B, 64), jnp.float32),
        ],
        scratch_shapes=[pltpu.VMEM((B, LANES), jnp.float32)],
    )(current_coords, tx, ty, spatial_features, W1, b1.reshape(1, 64))

    out = pl.pallas_call(
        _merge_body,
        in_specs=[
            pl.BlockSpec((B, 1), lambda: (0, 0)),
            pl.BlockSpec((B, 64), lambda: (0, 0)),
            pl.BlockSpec((NW, B), lambda: (0, 0)),
            pl.BlockSpec((B, 2), lambda: (0, 0)),
            pl.BlockSpec((129, 64), lambda: (0, 0)),
            pl.BlockSpec((64, 1), lambda: (0, 0)),
            pl.BlockSpec((1, 1), lambda: (0, 0)),
        ],
        out_specs=pl.BlockSpec((B, 1), lambda: (0, 0)),
        out_shape=jax.ShapeDtypeStruct((B, 1), jnp.float32),
    )(tcm, h0, scp, current_coords, W1, W2, b2.reshape(1, 1))
    return out


# speedup vs baseline: 1.0242x; 1.0242x over previous
import functools

import jax
import jax.numpy as jnp
from jax import lax
from jax.experimental import pallas as pl
from jax.experimental.pallas import tpu as pltpu
from jax.experimental.pallas import tpu_sc as plsc

B = 1024
KB = 7168
LANES = 128
BIG = 3.0e38

K_TOTAL = 100000
K_TC = 11 * KB
NW = 32
L = 16
NQ = 8
K_SC_REAL = K_TOTAL - K_TC
CH = -(-K_SC_REAL // (NW * L)) * L


def _sc_body(tx_hbm, ty_hbm, qx_hbm, qy_hbm, out_hbm, av, bv, cv, qxv, qyv,
             ov, tt):
    cid = lax.axis_index("c")
    sid = lax.axis_index("s")
    wid = sid * 2 + cid
    base = jnp.minimum(wid * CH, K_SC_REAL - CH)
    pltpu.sync_copy(tx_hbm.at[pl.ds(base, CH)], av)
    pltpu.sync_copy(ty_hbm.at[pl.ds(base, CH)], bv)
    pltpu.sync_copy(qx_hbm, qxv)
    pltpu.sync_copy(qy_hbm, qyv)

    def pre(i, carry):
        sl = pl.ds(i * L, L)
        tx = av[sl]
        ty = bv[sl]
        cv[sl] = tx * tx + ty * ty
        av[sl] = -2.0 * tx
        bv[sl] = -2.0 * ty
        return carry

    lax.fori_loop(0, CH // L, pre, 0, unroll=2)

    def qloop(qg, carry):
        qb = qg * L
        qxvec = qxv[pl.ds(qb, L)]
        qyvec = qyv[pl.ds(qb, L)]
        for sub in range(L // NQ):
            qxs = [qxvec[sub * NQ + u] for u in range(NQ)]
            qys = [qyvec[sub * NQ + u] for u in range(NQ)]

            def inner(j, accs):
                sl = pl.ds(j * L, L)
                a = av[sl]
                b = bv[sl]
                c = cv[sl]
                return tuple(
                    jnp.minimum(accs[u], c + qxs[u] * a + qys[u] * b)
                    for u in range(NQ))

            init = tuple(jnp.full((L,), BIG, jnp.float32) for _ in range(NQ))
            accs = lax.fori_loop(0, CH // L, inner, init, unroll=2)
            for u in range(NQ):
                tt[pl.ds((sub * NQ + u) * L, L)] = accs[u]
        rows = lax.iota(jnp.int32, L) * L
        mv = plsc.load_gather(tt, [rows])
        for j in range(1, L):
            mv = jnp.minimum(mv, plsc.load_gather(tt, [rows + j]))
        ov[pl.ds(qb, L)] = mv
        return carry

    lax.fori_loop(0, B // L, qloop, 0)
    pltpu.sync_copy(ov, out_hbm.at[wid])


def _sc_partial_min(tx_sc, ty_sc, qx, qy):
    mesh = plsc.VectorSubcoreMesh(
        core_axis_name="c", subcore_axis_name="s", num_cores=2,
        num_subcores=16)
    return pl.kernel(
        _sc_body,
        out_type=jax.ShapeDtypeStruct((NW, B), jnp.float32),
        mesh=mesh,
        compiler_params=pltpu.CompilerParams(needs_layout_passes=False),
        scratch_types=[
            pltpu.VMEM((CH,), jnp.float32),
            pltpu.VMEM((CH,), jnp.float32),
            pltpu.VMEM((CH,), jnp.float32),
            pltpu.VMEM((B,), jnp.float32),
            pltpu.VMEM((B,), jnp.float32),
            pltpu.VMEM((B,), jnp.float32),
            pltpu.VMEM((L * L,), jnp.float32),
        ],
    )(tx_sc, ty_sc, qx, qy)


def _tc_scan_body(cc_ref, tx_ref, ty_ref, out_ref, acc_ref, *, nsteps):
    i = pl.program_id(0)
    qx = cc_ref[:, 0:1]
    qy = cc_ref[:, 1:2]
    ax = -2.0 * qx
    ay = -2.0 * qy

    tx = tx_ref[:].reshape(1, KB)
    ty = ty_ref[:].reshape(1, KB)
    c = tx * tx + ty * ty
    ones = jnp.ones_like(qx)
    zeros8 = jnp.zeros((B, 5), jnp.float32)
    amat = jnp.concatenate([ax, ay, ones, zeros8], axis=1)
    tmat = jnp.concatenate(
        [tx, ty, c, jnp.zeros((5, KB), jnp.float32)], axis=0)
    p = jnp.dot(amat, tmat, preferred_element_type=jnp.float32,
                precision=lax.Precision.DEFAULT)

    m = p[:, 0:LANES]
    for g in range(1, KB // LANES):
        m = jnp.minimum(m, p[:, g * LANES:(g + 1) * LANES])

    @pl.when(i == 0)
    def _():
        acc_ref[...] = m

    @pl.when(i > 0)
    def _():
        acc_ref[...] = jnp.minimum(acc_ref[...], m)

    @pl.when(i == nsteps - 1)
    def _():
        out_ref[...] = jnp.min(acc_ref[...], axis=1, keepdims=True)


def _merge_body(tcm_ref, scp_ref, cc_ref, sf_ref, w1_ref, b1_ref, w2_ref,
                b2_ref, out_ref):
    qx = cc_ref[:, 0:1]
    qy = cc_ref[:, 1:2]
    q2 = qx * qx + qy * qy
    scm = jnp.min(jnp.transpose(scp_ref[...]), axis=1, keepdims=True)
    d2 = jnp.minimum(tcm_ref[...], scm) + q2
    dist = jnp.sqrt(jnp.maximum(d2, 1e-12))
    h = jnp.dot(sf_ref[...], w1_ref[0:128, :],
                preferred_element_type=jnp.float32)
    h = h + dist * w1_ref[128:129, :] + b1_ref[...]
    h = jnp.maximum(h, 0.0)
    z = jnp.dot(h, w2_ref[...], preferred_element_type=jnp.float32)
    out_ref[...] = jax.nn.sigmoid(z + b2_ref[...])


def kernel(spatial_features, current_coords, training_coords, W1, b1, W2, b2):
    scp = _sc_partial_min(training_coords[K_TC:, 0],
                          training_coords[K_TC:, 1],
                          current_coords[:, 0], current_coords[:, 1])

    tx = training_coords[:K_TC, 0]
    ty = training_coords[:K_TC, 1]

    nsteps = K_TC // KB
    tcm = pl.pallas_call(
        functools.partial(_tc_scan_body, nsteps=nsteps),
        grid=(nsteps,),
        in_specs=[
            pl.BlockSpec((B, 2), lambda i: (0, 0)),
            pl.BlockSpec((KB,), lambda i: (i,)),
            pl.BlockSpec((KB,), lambda i: (i,)),
        ],
        out_specs=pl.BlockSpec((B, 1), lambda i: (0, 0)),
        out_shape=jax.ShapeDtypeStruct((B, 1), jnp.float32),
        scratch_shapes=[pltpu.VMEM((B, LANES), jnp.float32)],
    )(current_coords, tx, ty)

    out = pl.pallas_call(
        _merge_body,
        in_specs=[
            pl.BlockSpec((B, 1), lambda: (0, 0)),
            pl.BlockSpec((NW, B), lambda: (0, 0)),
            pl.BlockSpec((B, 2), lambda: (0, 0)),
            pl.BlockSpec((B, 128), lambda: (0, 0)),
            pl.BlockSpec((129, 64), lambda: (0, 0)),
            pl.BlockSpec((1, 64), lambda: (0, 0)),
            pl.BlockSpec((64, 1), lambda: (0, 0)),
            pl.BlockSpec((1, 1), lambda: (0, 0)),
        ],
        out_specs=pl.BlockSpec((B, 1), lambda: (0, 0)),
        out_shape=jax.ShapeDtypeStruct((B, 1), jnp.float32),
    )(tcm, scp, current_coords, spatial_features,
      W1, b1.reshape(1, 64), W2, b2.reshape(1, 1))
    return out
